# baseline (speedup 1.0000x reference)
import functools

import jax
import jax.numpy as jnp
from jax import lax
from jax.experimental import pallas as pl
from jax.experimental.pallas import tpu as pltpu
from jax.experimental.pallas import tpu_sc as plsc

B = 16384
D = 16
V = 1000000
N_OUT = 1024
NC, NS = 2, 16
NW = NC * NS
B_PER_W = B // NW
CHUNK = 128
NCH = B_PER_W // CHUNK

_sc_mesh = plsc.VectorSubcoreMesh(core_axis_name="c", subcore_axis_name="s")


@functools.partial(
    pl.kernel,
    mesh=_sc_mesh,
    compiler_params=pltpu.CompilerParams(use_tc_tiling_on_sc=False),
    out_type=jax.ShapeDtypeStruct((D, B), jnp.float32),
    scratch_types=[
        pltpu.VMEM((NCH, CHUNK), jnp.int32),
        pltpu.VMEM((D, B_PER_W), jnp.float32),
        pltpu.SemaphoreType.DMA,
    ],
)
def _sc_gather(idx_hbm, table_hbm, out_hbm, idx_v, xt_v, sem):
    wid = lax.axis_index("s") * NC + lax.axis_index("c")
    pltpu.sync_copy(idx_hbm.at[wid], idx_v)
    copies = []
    for k in range(D):
        for j in range(NCH):
            copies.append(
                pltpu.async_copy(
                    table_hbm.at[k].at[idx_v.at[j]],
                    xt_v.at[k].at[pl.ds(j * CHUNK, CHUNK)],
                    sem,
                )
            )
    for cp in copies:
        cp.wait()
    pltpu.sync_copy(xt_v, out_hbm.at[:, pl.ds(wid * B_PER_W, B_PER_W)])


def _mm_body(w_ref, x_ref, b_ref, o_ref):
    o_ref[...] = (
        lax.dot_general(
            w_ref[...], x_ref[...], (((0,), (0,)), ((), ())),
            preferred_element_type=jnp.float32,
        )
        + b_ref[...]
    )


def _tc_matmul(w, x_t, b_col, block_m=1024):
    m = x_t.shape[1]
    return pl.pallas_call(
        _mm_body,
        grid=(m // block_m,),
        in_specs=[
            pl.BlockSpec((D, N_OUT), lambda i: (0, 0)),
            pl.BlockSpec((D, block_m), lambda i: (0, i)),
            pl.BlockSpec((N_OUT, 1), lambda i: (0, 0)),
        ],
        out_specs=pl.BlockSpec((N_OUT, block_m), lambda i: (0, i)),
        out_shape=jax.ShapeDtypeStruct((N_OUT, m), jnp.float32),
    )(w, x_t, b_col)


def kernel(inputs, emb_table, dense_w, dense_b):
    idx = inputs.reshape(NW, NCH, CHUNK).astype(jnp.int32)
    x_t = _sc_gather(idx, emb_table.T)
    out_t = _tc_matmul(dense_w, x_t, dense_b.reshape(N_OUT, 1))
    return out_t.T.reshape(B, 4, 4, 64)


# speedup vs baseline: 2.6497x; 2.6497x over previous
import functools

import jax
import jax.numpy as jnp
from jax import lax
from jax.experimental import pallas as pl
from jax.experimental.pallas import tpu as pltpu
from jax.experimental.pallas import tpu_sc as plsc

B = 16384
D = 16
V = 1000000
N_OUT = 1024
NC, NS = 2, 16
NW = NC * NS
B_PER_W = B // NW
CHUNK = 128
NCH = B_PER_W // CHUNK
L = 16

_sc_mesh = plsc.VectorSubcoreMesh(core_axis_name="c", subcore_axis_name="s")


def _detile_body(src_ref, dst_ref, sem):
    copies = [
        pltpu.make_async_copy(src_ref.at[k], dst_ref.at[pl.ds(k * V, V)], sem)
        for k in range(D)
    ]
    for cp in copies:
        cp.start()
    for cp in copies:
        cp.wait()


def _detile(table_t):
    return pl.pallas_call(
        _detile_body,
        in_specs=[pl.BlockSpec(memory_space=pl.ANY)],
        out_specs=pl.BlockSpec(memory_space=pl.ANY),
        out_shape=jax.ShapeDtypeStruct((D * V,), jnp.float32),
        scratch_shapes=[pltpu.SemaphoreType.DMA],
    )(table_t)


@functools.partial(
    pl.kernel,
    mesh=_sc_mesh,
    compiler_params=pltpu.CompilerParams(use_tc_tiling_on_sc=False),
    out_type=jax.ShapeDtypeStruct((D, B), jnp.float32),
    scratch_types=[
        pltpu.VMEM((NCH, CHUNK), jnp.int32),
        pltpu.VMEM((D * NCH, CHUNK), jnp.int32),
        pltpu.VMEM((D, B_PER_W), jnp.float32),
        pltpu.SemaphoreType.DMA,
    ],
)
def _sc_gather(idx_hbm, table_hbm, out_hbm, idx_v, idxs_v, xt_v, sem):
    wid = lax.axis_index("s") * NC + lax.axis_index("c")
    pltpu.sync_copy(idx_hbm.at[wid], idx_v)

    def group(g, _):
        base = g * L
        for j in range(NCH):
            iv = idx_v[j, pl.ds(base, L)]
            for k in range(D):
                idxs_v[k * NCH + j, pl.ds(base, L)] = iv * D + k
        return 0

    lax.fori_loop(0, CHUNK // L, group, 0)

    copies = []
    for k in range(D):
        for j in range(NCH):
            copies.append(
                pltpu.async_copy(
                    table_hbm.at[idxs_v.at[k * NCH + j]],
                    xt_v.at[k].at[pl.ds(j * CHUNK, CHUNK)],
                    sem,
                )
            )
    for cp in copies:
        cp.wait()
    pltpu.sync_copy(xt_v, out_hbm.at[:, pl.ds(wid * B_PER_W, B_PER_W)])


def _mm_body(w_ref, x_ref, b_ref, o_ref):
    o_ref[...] = (
        lax.dot_general(
            w_ref[...], x_ref[...], (((0,), (0,)), ((), ())),
            preferred_element_type=jnp.float32,
        )
        + b_ref[...]
    )


def _tc_matmul(w, x_t, b_col, block_m=1024):
    m = x_t.shape[1]
    return pl.pallas_call(
        _mm_body,
        grid=(m // block_m,),
        in_specs=[
            pl.BlockSpec((D, N_OUT), lambda i: (0, 0)),
            pl.BlockSpec((D, block_m), lambda i: (0, i)),
            pl.BlockSpec((N_OUT, 1), lambda i: (0, 0)),
        ],
        out_specs=pl.BlockSpec((N_OUT, block_m), lambda i: (0, i)),
        out_shape=jax.ShapeDtypeStruct((N_OUT, m), jnp.float32),
    )(w, x_t, b_col)


def kernel(inputs, emb_table, dense_w, dense_b):
    idx = inputs.reshape(NW, NCH, CHUNK).astype(jnp.int32)
    x_t = _sc_gather(idx, emb_table.reshape(D * V))
    out_t = _tc_matmul(dense_w, x_t, dense_b.reshape(N_OUT, 1))
    return out_t.T.reshape(B, 4, 4, 64)


# speedup vs baseline: 2.6742x; 1.0092x over previous
import functools

import jax
import jax.numpy as jnp
from jax import lax
from jax.experimental import pallas as pl
from jax.experimental.pallas import tpu as pltpu
from jax.experimental.pallas import tpu_sc as plsc

B = 16384
D = 16
V = 1000000
N_OUT = 1024
NC, NS = 2, 16
NW = NC * NS
B_PER_W = B // NW
CHUNK = 128
NCH = B_PER_W // CHUNK

_sc_mesh = plsc.VectorSubcoreMesh(core_axis_name="c", subcore_axis_name="s")


@functools.partial(
    pl.kernel,
    mesh=_sc_mesh,
    compiler_params=pltpu.CompilerParams(use_tc_tiling_on_sc=False),
    out_type=jax.ShapeDtypeStruct((NW, NCH, CHUNK, D), jnp.float32),
    scratch_types=[
        pltpu.VMEM((NCH, CHUNK), jnp.int32),
        pltpu.VMEM((NCH, CHUNK, D), jnp.float32),
        pltpu.SemaphoreType.DMA,
    ],
)
def _sc_gather(idx_hbm, table_hbm, out_hbm, idx_v, rows_v, sem):
    wid = lax.axis_index("s") * NC + lax.axis_index("c")
    pltpu.sync_copy(idx_hbm.at[wid], idx_v)
    copies = []
    for j in range(NCH):
        copies.append(
            pltpu.async_copy(table_hbm.at[idx_v.at[j]], rows_v.at[j], sem)
        )
    for cp in copies:
        cp.wait()
    pltpu.sync_copy(rows_v, out_hbm.at[wid])


def _mm_body(w_ref, x_ref, b_ref, o_ref):
    o_ref[...] = (
        lax.dot_general(
            w_ref[...], x_ref[...], (((0,), (1,)), ((), ())),
            preferred_element_type=jnp.float32,
        )
        + b_ref[...]
    )


def _tc_matmul(w, x, b_col, block_m=1024):
    m = x.shape[0]
    return pl.pallas_call(
        _mm_body,
        grid=(m // block_m,),
        in_specs=[
            pl.BlockSpec((D, N_OUT), lambda i: (0, 0)),
            pl.BlockSpec((block_m, D), lambda i: (i, 0)),
            pl.BlockSpec((N_OUT, 1), lambda i: (0, 0)),
        ],
        out_specs=pl.BlockSpec((N_OUT, block_m), lambda i: (0, i)),
        out_shape=jax.ShapeDtypeStruct((N_OUT, m), jnp.float32),
    )(w, x, b_col)


def kernel(inputs, emb_table, dense_w, dense_b):
    idx = jnp.minimum(inputs.reshape(NW, NCH, CHUNK).astype(jnp.int32), V - 1)
    rows = _sc_gather(idx, emb_table).reshape(B, D)
    out_t = _tc_matmul(dense_w, rows, dense_b.reshape(N_OUT, 1))
    return out_t.T.reshape(B, 4, 4, 64)
